# Initial kernel scaffold; baseline (speedup 1.0000x reference)
import jax
import jax.numpy as jnp
from jax.experimental import pallas as pl


def kernel(tree_x, edge_index, batch, W_g1, b_g1, W_g2, b_g2, W_g3, b_g3, W_f1, b_f1, W_f2, b_f2, W_f3, b_f3, W_out, b_out):
    raise NotImplementedError("write your pallas kernel here")


# speedup vs baseline: 1.0000x; 1.0000x over previous
import jax
import jax.numpy as jnp
from jax.experimental import pallas as pl

N_NODES = 10000
N_GRAPHS = 64


def _head_body(x_ref, w1, b1, w2, b2, w3, b3, wo, bo, out_ref):
    x = x_ref[...]
    x = jax.nn.relu(jnp.dot(x, w1[...], preferred_element_type=jnp.float32) + b1[...])
    x = jax.nn.relu(jnp.dot(x, w2[...], preferred_element_type=jnp.float32) + b2[...])
    x = jax.nn.relu(jnp.dot(x, w3[...], preferred_element_type=jnp.float32) + b3[...])
    out_ref[...] = jnp.dot(x, wo[...], preferred_element_type=jnp.float32) + bo[...]


def _gcn_conv(x, edge_index, W, b, n_nodes):
    src = edge_index[0]
    dst = edge_index[1]
    loop = jnp.arange(n_nodes, dtype=src.dtype)
    src = jnp.concatenate([src, loop])
    dst = jnp.concatenate([dst, loop])
    deg = jax.ops.segment_sum(jnp.ones_like(dst, dtype=x.dtype), dst, num_segments=n_nodes)
    dinv = jnp.where(deg > 0, jax.lax.rsqrt(deg), 0.0)
    norm = dinv[src] * dinv[dst]
    h = x @ W
    msg = h[src] * norm[:, None]
    out = jax.ops.segment_sum(msg, dst, num_segments=n_nodes)
    return out + b


def kernel(tree_x, edge_index, batch, W_g1, b_g1, W_g2, b_g2, W_g3, b_g3,
           W_f1, b_f1, W_f2, b_f2, W_f3, b_f3, W_out, b_out):
    h = jax.nn.relu(_gcn_conv(tree_x, edge_index, W_g1, b_g1, N_NODES))
    h = jax.nn.relu(_gcn_conv(h, edge_index, W_g2, b_g2, N_NODES))
    h = jax.nn.relu(_gcn_conv(h, edge_index, W_g3, b_g3, N_NODES))
    x = jax.ops.segment_max(h, batch, num_segments=N_GRAPHS)
    out = pl.pallas_call(
        _head_body,
        out_shape=jax.ShapeDtypeStruct((N_GRAPHS, W_out.shape[1]), jnp.float32),
    )(x, W_f1, b_f1, W_f2, b_f2, W_f3, b_f3, W_out, b_out)
    return out


# speedup vs baseline: 6.3013x; 6.3012x over previous
import functools
import jax
import jax.numpy as jnp
from jax import lax
from jax.experimental import pallas as pl
from jax.experimental.pallas import tpu as pltpu
from jax.experimental.pallas import tpu_sc as plsc

N_NODES = 10000
N_EDGES = 320000
N_GRAPHS = 64
D_IN = 128
D_H = 512
N_ACTIONS = 32

NC = 2
NS = 16
EB = 128
NB = 160
E_PER_TILE = NB * EB
E_PAD = NS * E_PER_TILE
N_PAD = 10240
ACC_ROWS = N_PAD
ROWS_PER_TILE = N_PAD // NS

_MESH = plsc.VectorSubcoreMesh(core_axis_name="c", subcore_axis_name="s")


HB = NB // 2


def _deg_body(dst_hbm, ones_hbm, d0_hbm, d1_hbm, di, ones_v, acc):
    c = lax.axis_index("c")
    s = lax.axis_index("s")
    pltpu.sync_copy(dst_hbm.at[pl.ds((c * NS + s) * HB, HB)], di)
    pltpu.sync_copy(ones_hbm.at[pl.ds(0, EB)], ones_v)
    pltpu.sync_copy(ones_hbm.at[pl.ds(s * ROWS_PER_TILE, ROWS_PER_TILE)],
                    acc.at[pl.ds(s * ROWS_PER_TILE, ROWS_PER_TILE)])
    plsc.subcore_barrier()

    def step(b, _):
        pltpu.sync_copy(ones_v, acc.at[di.at[b]], add=True)
        return 0

    lax.fori_loop(0, HB, step, 0)
    plsc.subcore_barrier()
    for cc in range(NC):
        @pl.when(c == cc)
        def _(cc=cc):
            out = (d0_hbm, d1_hbm)[cc]
            pltpu.sync_copy(acc.at[pl.ds(s * ROWS_PER_TILE, ROWS_PER_TILE)],
                            out.at[pl.ds(s * ROWS_PER_TILE, ROWS_PER_TILE)])


_deg_call = functools.partial(
    pl.kernel,
    out_type=[jax.ShapeDtypeStruct((N_PAD, EB), jnp.float32)] * 2,
    mesh=_MESH,
    scratch_types=[
        pltpu.VMEM((HB, EB), jnp.int32),
        pltpu.VMEM((EB, EB), jnp.float32),
        pltpu.VMEM_SHARED((ACC_ROWS, EB), jnp.float32),
    ],
)(_deg_body)


IB = 16


def _edge_scan(feat_hbm, src_hbm, dst_hbm, base, nb, si, di,
               bufA, bufB, acc, semA, semB):
    def blk(q, _):
        row0 = base + q * IB
        pltpu.sync_copy(src_hbm.at[pl.ds(row0, IB)], si)
        pltpu.sync_copy(dst_hbm.at[pl.ds(row0, IB)], di)
        pltpu.async_copy(feat_hbm.at[si.at[0]], bufA, semA)
        pltpu.async_copy(feat_hbm.at[si.at[1]], bufB, semB)
        for j in range(IB // 2):
            pltpu.make_async_copy(feat_hbm.at[si.at[2 * j]], bufA, semA).wait()
            pltpu.sync_copy(bufA, acc.at[di.at[2 * j]], add=True)
            if j < IB // 2 - 1:
                pltpu.async_copy(feat_hbm.at[si.at[2 * j + 2]], bufA, semA)
            pltpu.make_async_copy(feat_hbm.at[si.at[2 * j + 1]], bufB, semB).wait()
            pltpu.sync_copy(bufB, acc.at[di.at[2 * j + 1]], add=True)
            if j < IB // 2 - 1:
                pltpu.async_copy(feat_hbm.at[si.at[2 * j + 3]], bufB, semB)
        return 0

    lax.fori_loop(0, nb // IB, blk, 0)


def _seed(src_hbm, dst_ref, s):
    pltpu.sync_copy(src_hbm.at[pl.ds(s * ROWS_PER_TILE, ROWS_PER_TILE)],
                    dst_ref.at[pl.ds(s * ROWS_PER_TILE, ROWS_PER_TILE)])


def _writeout(acc, out_hbm, s):
    pltpu.sync_copy(acc.at[pl.ds(s * ROWS_PER_TILE, ROWS_PER_TILE)],
                    out_hbm.at[pl.ds(s * ROWS_PER_TILE, ROWS_PER_TILE)])


_AGG_SCRATCH = [
    pltpu.VMEM((IB, EB), jnp.int32),
    pltpu.VMEM((IB, EB), jnp.int32),
    pltpu.VMEM((EB, EB), jnp.float32),
    pltpu.VMEM((EB, EB), jnp.float32),
    pltpu.VMEM_SHARED((ACC_ROWS, EB), jnp.float32),
    pltpu.SemaphoreType.DMA,
    pltpu.SemaphoreType.DMA,
]


def _agg128_body(src_hbm, dst_hbm, z0, z1, z2, z3, o0, o1, o2, o3,
                 si, di, bufA, bufB, acc, semA, semB):
    c = lax.axis_index("c")
    s = lax.axis_index("s")
    ins = (z0, z1, z2, z3)
    outs = (o0, o1, o2, o3)
    for cc in range(NC):
        @pl.when(c == cc)
        def _(cc=cc):
            for k in range(2):
                a = cc * 2 + k
                _seed(ins[a], acc, s)
                plsc.subcore_barrier()
                _edge_scan(ins[a], src_hbm, dst_hbm, s * NB, NB,
                           si, di, bufA, bufB, acc, semA, semB)
                plsc.subcore_barrier()
                _writeout(acc, outs[a], s)
                plsc.subcore_barrier()


_agg128 = functools.partial(
    pl.kernel,
    out_type=[jax.ShapeDtypeStruct((N_PAD, EB), jnp.float32)] * 4,
    mesh=_MESH,
    scratch_types=_AGG_SCRATCH,
)(_agg128_body)


def _agg1_body(src_hbm, dst_hbm, z_hbm, o0_hbm, o1_hbm,
               si, di, bufA, bufB, acc, semA, semB):
    c = lax.axis_index("c")
    s = lax.axis_index("s")
    _seed(z_hbm, acc, s)
    plsc.subcore_barrier()
    _edge_scan(z_hbm, src_hbm, dst_hbm, (c * NS + s) * HB, HB,
               si, di, bufA, bufB, acc, semA, semB)
    plsc.subcore_barrier()
    for cc in range(NC):
        @pl.when(c == cc)
        def _(cc=cc):
            _writeout(acc, (o0_hbm, o1_hbm)[cc], s)


_agg1 = functools.partial(
    pl.kernel,
    out_type=[jax.ShapeDtypeStruct((N_PAD, EB), jnp.float32)] * 2,
    mesh=_MESH,
    scratch_types=_AGG_SCRATCH,
)(_agg1_body)


_BM = 1024


def _prescale_body(d0_ref, d1_ref, x_ref, dinv_ref, z_ref):
    deg = d0_ref[...] + d1_ref[...] - 1.0
    dinv = lax.rsqrt(deg)
    dinv_ref[...] = dinv
    z_ref[...] = x_ref[...] * dinv


def _mm_mid_body(dinv_ref, a0, a1, a2, a3, w_ref, b_ref, o0, o1, o2, o3):
    dinv = dinv_ref[...]
    agg = jnp.concatenate([a0[...], a1[...], a2[...], a3[...]], axis=1)
    h = jnp.maximum(
        jnp.dot(agg * dinv, w_ref[...], preferred_element_type=jnp.float32)
        + b_ref[...], 0.0) * dinv
    o0[...] = h[:, 0:128]
    o1[...] = h[:, 128:256]
    o2[...] = h[:, 256:384]
    o3[...] = h[:, 384:512]


def _mm1_body(dinv_ref, a0_ref, a1_ref, z_ref, w_ref, b_ref, o0, o1, o2, o3):
    dinv = dinv_ref[...]
    agg = a0_ref[...] + a1_ref[...] - z_ref[...]
    h = jnp.maximum(
        jnp.dot(agg * dinv, w_ref[...], preferred_element_type=jnp.float32)
        + b_ref[...], 0.0) * dinv
    o0[...] = h[:, 0:128]
    o1[...] = h[:, 128:256]
    o2[...] = h[:, 256:384]
    o3[...] = h[:, 384:512]


def _mm3_body(dinv_ref, a0, a1, a2, a3, w_ref, b_ref, o_ref):
    dinv = dinv_ref[...]
    agg = jnp.concatenate([a0[...], a1[...], a2[...], a3[...]], axis=1)
    o_ref[...] = jnp.maximum(
        jnp.dot(agg * dinv, w_ref[...], preferred_element_type=jnp.float32)
        + b_ref[...], 0.0)


def _head_body(x_ref, w1, b1, w2, b2, w3, b3, wo, bo, out_ref):
    x = x_ref[...]
    x = jnp.maximum(jnp.dot(x, w1[...], preferred_element_type=jnp.float32) + b1[...], 0.0)
    x = jnp.maximum(jnp.dot(x, w2[...], preferred_element_type=jnp.float32) + b2[...], 0.0)
    x = jnp.maximum(jnp.dot(x, w3[...], preferred_element_type=jnp.float32) + b3[...], 0.0)
    out_ref[...] = jnp.dot(x, wo[...], preferred_element_type=jnp.float32) + bo[...]


def _row_spec(w):
    return pl.BlockSpec((_BM, w), lambda i: (i, 0))


def _rep_spec(shape):
    return pl.BlockSpec(shape, lambda i: (0, 0))


def kernel(tree_x, edge_index, batch, W_g1, b_g1, W_g2, b_g2, W_g3, b_g3,
           W_f1, b_f1, W_f2, b_f2, W_f3, b_f3, W_out, b_out):
    src = edge_index[0]
    dst = edge_index[1]
    pad = E_PAD - N_EDGES
    src_p = jnp.concatenate([src, jnp.zeros((pad,), jnp.int32)]).reshape(NS * NB, EB)
    dst_p = jnp.concatenate([dst, jnp.full((pad,), N_NODES, jnp.int32)]).reshape(NS * NB, EB)
    ones_seed = jnp.ones((ACC_ROWS, EB), jnp.float32)
    d0, d1 = _deg_call(dst_p, ones_seed)
    x_pad = jnp.pad(tree_x, ((0, N_PAD - N_NODES), (0, 0)))

    dinv, z1 = pl.pallas_call(
        _prescale_body,
        grid=(N_PAD // _BM,),
        in_specs=[_row_spec(1), _row_spec(1), _row_spec(D_IN)],
        out_specs=[_row_spec(1), _row_spec(D_IN)],
        out_shape=[
            jax.ShapeDtypeStruct((N_PAD, 1), jnp.float32),
            jax.ShapeDtypeStruct((N_PAD, D_IN), jnp.float32),
        ],
    )(d0[:, :1], d1[:, :1], x_pad)

    g1a, g1b = _agg1(src_p, dst_p, z1)

    z2 = pl.pallas_call(
        _mm1_body,
        grid=(N_PAD // _BM,),
        in_specs=[_row_spec(1), _row_spec(128), _row_spec(128), _row_spec(128),
                  _rep_spec((D_IN, D_H)), _rep_spec((1, D_H))],
        out_specs=[_row_spec(128)] * 4,
        out_shape=[jax.ShapeDtypeStruct((N_PAD, 128), jnp.float32)] * 4,
    )(dinv, g1a, g1b, z1, W_g1, b_g1.reshape(1, D_H))

    g2 = _agg128(src_p, dst_p, *z2)

    def mm_mid(chunks, W, b):
        return pl.pallas_call(
            _mm_mid_body,
            grid=(N_PAD // _BM,),
            in_specs=[_row_spec(1)] + [_row_spec(128)] * 4
                     + [_rep_spec((D_H, D_H)), _rep_spec((1, D_H))],
            out_specs=[_row_spec(128)] * 4,
            out_shape=[jax.ShapeDtypeStruct((N_PAD, 128), jnp.float32)] * 4,
        )(dinv, *chunks, W, b.reshape(1, D_H))

    z3 = mm_mid(g2, W_g2, b_g2)
    g3 = _agg128(src_p, dst_p, *z3)

    h3 = pl.pallas_call(
        _mm3_body,
        grid=(N_PAD // _BM,),
        in_specs=[_row_spec(1)] + [_row_spec(128)] * 4
                 + [_rep_spec((D_H, D_H)), _rep_spec((1, D_H))],
        out_specs=_row_spec(D_H),
        out_shape=jax.ShapeDtypeStruct((N_PAD, D_H), jnp.float32),
    )(dinv, *g3, W_g3, b_g3.reshape(1, D_H))

    pooled = jax.ops.segment_max(h3[:N_NODES], batch, num_segments=N_GRAPHS)

    return pl.pallas_call(
        _head_body,
        out_shape=jax.ShapeDtypeStruct((N_GRAPHS, N_ACTIONS), jnp.float32),
    )(pooled, W_f1, b_f1.reshape(1, D_H), W_f2, b_f2.reshape(1, D_H),
      W_f3, b_f3.reshape(1, D_H), W_out, b_out.reshape(1, N_ACTIONS))


# speedup vs baseline: 6.7933x; 1.0781x over previous
import functools
import jax
import jax.numpy as jnp
from jax import lax
from jax.experimental import pallas as pl
from jax.experimental.pallas import tpu as pltpu
from jax.experimental.pallas import tpu_sc as plsc

N_NODES = 10000
N_EDGES = 320000
N_GRAPHS = 64
D_IN = 128
D_H = 512
N_ACTIONS = 32

NC = 2
NS = 16
EB = 128
NB = 160
E_PER_TILE = NB * EB
E_PAD = NS * E_PER_TILE
N_PAD = 10240
ACC_ROWS = N_PAD
ROWS_PER_TILE = N_PAD // NS

_MESH = plsc.VectorSubcoreMesh(core_axis_name="c", subcore_axis_name="s")


HB = NB // 2


def _deg_body(dst_hbm, ones_hbm, d0_hbm, d1_hbm, di, ones_v, acc):
    c = lax.axis_index("c")
    s = lax.axis_index("s")
    pltpu.sync_copy(dst_hbm.at[pl.ds((c * NS + s) * HB, HB)], di)
    pltpu.sync_copy(ones_hbm.at[pl.ds(0, EB)], ones_v)
    pltpu.sync_copy(ones_hbm.at[pl.ds(s * ROWS_PER_TILE, ROWS_PER_TILE)],
                    acc.at[pl.ds(s * ROWS_PER_TILE, ROWS_PER_TILE)])
    plsc.subcore_barrier()

    def step(b, _):
        pltpu.sync_copy(ones_v, acc.at[di.at[b]], add=True)
        return 0

    lax.fori_loop(0, HB, step, 0)
    plsc.subcore_barrier()
    for cc in range(NC):
        @pl.when(c == cc)
        def _(cc=cc):
            out = (d0_hbm, d1_hbm)[cc]
            pltpu.sync_copy(acc.at[pl.ds(s * ROWS_PER_TILE, ROWS_PER_TILE)],
                            out.at[pl.ds(s * ROWS_PER_TILE, ROWS_PER_TILE)])


_deg_call = functools.partial(
    pl.kernel,
    out_type=[jax.ShapeDtypeStruct((N_PAD, EB), jnp.float32)] * 2,
    mesh=_MESH,
    scratch_types=[
        pltpu.VMEM((HB, EB), jnp.int32),
        pltpu.VMEM((EB, EB), jnp.float32),
        pltpu.VMEM_SHARED((ACC_ROWS, EB), jnp.float32),
    ],
)(_deg_body)


IB = 16


def _edge_scan(feat_hbm, src_hbm, dst_hbm, base, nb, si, di,
               bufA, bufB, acc, semA, semB, semSA, semSB):
    def blk(q, _):
        row0 = base + q * IB
        pltpu.sync_copy(src_hbm.at[pl.ds(row0, IB)], si)
        pltpu.sync_copy(dst_hbm.at[pl.ds(row0, IB)], di)
        pltpu.async_copy(feat_hbm.at[si.at[0]], bufA, semA)
        pltpu.async_copy(feat_hbm.at[si.at[1]], bufB, semB)
        for j in range(IB // 2):
            pltpu.make_async_copy(feat_hbm.at[si.at[2 * j]], bufA, semA).wait()
            pltpu.async_copy(bufA, acc.at[di.at[2 * j]], semSA, add=True)
            pltpu.make_async_copy(feat_hbm.at[si.at[2 * j + 1]], bufB, semB).wait()
            pltpu.async_copy(bufB, acc.at[di.at[2 * j + 1]], semSB, add=True)
            if j < IB // 2 - 1:
                pltpu.make_async_copy(bufA, acc.at[di.at[2 * j]], semSA).wait()
                pltpu.async_copy(feat_hbm.at[si.at[2 * j + 2]], bufA, semA)
                pltpu.make_async_copy(bufB, acc.at[di.at[2 * j + 1]], semSB).wait()
                pltpu.async_copy(feat_hbm.at[si.at[2 * j + 3]], bufB, semB)
            else:
                pltpu.make_async_copy(bufA, acc.at[di.at[2 * j]], semSA).wait()
                pltpu.make_async_copy(bufB, acc.at[di.at[2 * j + 1]], semSB).wait()
        return 0

    lax.fori_loop(0, nb // IB, blk, 0)


def _seed(src_hbm, dst_ref, s):
    pltpu.sync_copy(src_hbm.at[pl.ds(s * ROWS_PER_TILE, ROWS_PER_TILE)],
                    dst_ref.at[pl.ds(s * ROWS_PER_TILE, ROWS_PER_TILE)])


def _writeout(acc, out_hbm, s):
    pltpu.sync_copy(acc.at[pl.ds(s * ROWS_PER_TILE, ROWS_PER_TILE)],
                    out_hbm.at[pl.ds(s * ROWS_PER_TILE, ROWS_PER_TILE)])


_AGG_SCRATCH = [
    pltpu.VMEM((IB, EB), jnp.int32),
    pltpu.VMEM((IB, EB), jnp.int32),
    pltpu.VMEM((EB, EB), jnp.float32),
    pltpu.VMEM((EB, EB), jnp.float32),
    pltpu.VMEM_SHARED((ACC_ROWS, EB), jnp.float32),
    pltpu.SemaphoreType.DMA,
    pltpu.SemaphoreType.DMA,
    pltpu.SemaphoreType.DMA,
    pltpu.SemaphoreType.DMA,
]


def _agg128_body(src_hbm, dst_hbm, z0, z1, z2, z3, o0, o1, o2, o3,
                 si, di, bufA, bufB, acc, semA, semB, semSA, semSB):
    c = lax.axis_index("c")
    s = lax.axis_index("s")
    ins = (z0, z1, z2, z3)
    outs = (o0, o1, o2, o3)
    for cc in range(NC):
        @pl.when(c == cc)
        def _(cc=cc):
            for k in range(2):
                a = cc * 2 + k
                _seed(ins[a], acc, s)
                plsc.subcore_barrier()
                _edge_scan(ins[a], src_hbm, dst_hbm, s * NB, NB,
                           si, di, bufA, bufB, acc, semA, semB, semSA, semSB)
                plsc.subcore_barrier()
                _writeout(acc, outs[a], s)
                plsc.subcore_barrier()


_agg128 = functools.partial(
    pl.kernel,
    out_type=[jax.ShapeDtypeStruct((N_PAD, EB), jnp.float32)] * 4,
    mesh=_MESH,
    scratch_types=_AGG_SCRATCH,
)(_agg128_body)


def _agg1_body(src_hbm, dst_hbm, z_hbm, o0_hbm, o1_hbm,
               si, di, bufA, bufB, acc, semA, semB, semSA, semSB):
    c = lax.axis_index("c")
    s = lax.axis_index("s")
    _seed(z_hbm, acc, s)
    plsc.subcore_barrier()
    _edge_scan(z_hbm, src_hbm, dst_hbm, (c * NS + s) * HB, HB,
               si, di, bufA, bufB, acc, semA, semB, semSA, semSB)
    plsc.subcore_barrier()
    for cc in range(NC):
        @pl.when(c == cc)
        def _(cc=cc):
            _writeout(acc, (o0_hbm, o1_hbm)[cc], s)


_agg1 = functools.partial(
    pl.kernel,
    out_type=[jax.ShapeDtypeStruct((N_PAD, EB), jnp.float32)] * 2,
    mesh=_MESH,
    scratch_types=_AGG_SCRATCH,
)(_agg1_body)


G_ACC = 72
SB = 64
NSB = ROWS_PER_TILE // SB
_I32MIN = -2147483648


def _pool_body(c00, c01, c10, c11, batch_hbm, out0, out1,
               b_v, r0, r1, acc, tmp, res, spm):
    c = lax.axis_index("c")
    s = lax.axis_index("s")
    pltpu.sync_copy(batch_hbm.at[pl.ds(s * ROWS_PER_TILE, ROWS_PER_TILE)], b_v)
    neg = jnp.full((16,), -jnp.inf, jnp.float32)

    def ini(i, _):
        acc[pl.ds(i * 16, 16)] = neg
        return 0

    lax.fori_loop(0, G_ACC * 256 // 16, ini, 0)

    for cc in range(NC):
        @pl.when(c == cc)
        def _(cc=cc):
            h0, h1 = ((c00, c01), (c10, c11))[cc]

            def sbloop(sb, _):
                row0 = s * ROWS_PER_TILE + sb * SB
                pltpu.sync_copy(h0.at[pl.ds(row0, SB)], r0)
                pltpu.sync_copy(h1.at[pl.ds(row0, SB)], r1)
                for grp in range(SB // 16):
                    bv = b_v[pl.ds(sb * SB + grp * 16, 16)]
                    for lane in range(16):
                        r = grp * 16 + lane
                        g = bv[lane]
                        base = g * 256
                        for k in range(8):
                            o = base + k * 16
                            acc[pl.ds(o, 16)] = jnp.maximum(
                                acc[pl.ds(o, 16)], r0[r, pl.ds(k * 16, 16)])
                        for k in range(8):
                            o = base + 128 + k * 16
                            acc[pl.ds(o, 16)] = jnp.maximum(
                                acc[pl.ds(o, 16)], r1[r, pl.ds(k * 16, 16)])
                return 0

            lax.fori_loop(0, NSB, sbloop, 0)
    pltpu.sync_copy(acc, spm.at[s])
    plsc.subcore_barrier()

    @pl.when(s < 8)
    def _():
        def rini(i, _):
            res[pl.ds(i * 16, 16)] = neg
            return 0

        lax.fori_loop(0, 2048 // 16, rini, 0)
        for s2 in range(NS):
            pltpu.sync_copy(spm.at[s2, pl.ds(s * 2048, 2048)], tmp)

            def mx(i, _):
                res[pl.ds(i * 16, 16)] = jnp.maximum(
                    res[pl.ds(i * 16, 16)], tmp[pl.ds(i * 16, 16)])
                return 0

            lax.fori_loop(0, 2048 // 16, mx, 0)
        for cc in range(NC):
            @pl.when(c == cc)
            def _(cc=cc):
                out = (out0, out1)[cc]
                pltpu.sync_copy(res, out.at[pl.ds(s * 2048, 2048)])


_pool_call = functools.partial(
    pl.kernel,
    out_type=[jax.ShapeDtypeStruct((N_GRAPHS * 256,), jnp.float32)] * 2,
    mesh=_MESH,
    scratch_types=[
        pltpu.VMEM((ROWS_PER_TILE,), jnp.int32),
        pltpu.VMEM((SB, EB), jnp.float32),
        pltpu.VMEM((SB, EB), jnp.float32),
        pltpu.VMEM((G_ACC * 256,), jnp.float32),
        pltpu.VMEM((2048,), jnp.float32),
        pltpu.VMEM((2048,), jnp.float32),
        pltpu.VMEM_SHARED((NS, G_ACC * 256), jnp.float32),
    ],
)(_pool_body)


_BM = 1024


def _prescale_body(d0_ref, d1_ref, x_ref, dinv_ref, z_ref):
    deg = d0_ref[...] + d1_ref[...] - 1.0
    dinv = lax.rsqrt(deg)
    dinv_ref[...] = dinv
    z_ref[...] = x_ref[...] * dinv


def _mm_mid_body(dinv_ref, a0, a1, a2, a3, w_ref, b_ref, o0, o1, o2, o3):
    dinv = dinv_ref[...]
    agg = jnp.concatenate([a0[...], a1[...], a2[...], a3[...]], axis=1)
    h = jnp.maximum(
        jnp.dot(agg * dinv, w_ref[...], preferred_element_type=jnp.float32)
        + b_ref[...], 0.0) * dinv
    o0[...] = h[:, 0:128]
    o1[...] = h[:, 128:256]
    o2[...] = h[:, 256:384]
    o3[...] = h[:, 384:512]


def _mm1_body(dinv_ref, a0_ref, a1_ref, z_ref, w_ref, b_ref, o0, o1, o2, o3):
    dinv = dinv_ref[...]
    agg = a0_ref[...] + a1_ref[...] - z_ref[...]
    h = jnp.maximum(
        jnp.dot(agg * dinv, w_ref[...], preferred_element_type=jnp.float32)
        + b_ref[...], 0.0) * dinv
    o0[...] = h[:, 0:128]
    o1[...] = h[:, 128:256]
    o2[...] = h[:, 256:384]
    o3[...] = h[:, 384:512]


def _mm3_body(dinv_ref, a0, a1, a2, a3, w_ref, b_ref, o0, o1, o2, o3):
    dinv = dinv_ref[...]
    agg = jnp.concatenate([a0[...], a1[...], a2[...], a3[...]], axis=1)
    h = jnp.maximum(
        jnp.dot(agg * dinv, w_ref[...], preferred_element_type=jnp.float32)
        + b_ref[...], 0.0)
    o0[...] = h[:, 0:128]
    o1[...] = h[:, 128:256]
    o2[...] = h[:, 256:384]
    o3[...] = h[:, 384:512]


def _head_body(x_ref, w1, b1, w2, b2, w3, b3, wo, bo, out_ref):
    x = x_ref[...]
    x = jnp.maximum(jnp.dot(x, w1[...], preferred_element_type=jnp.float32) + b1[...], 0.0)
    x = jnp.maximum(jnp.dot(x, w2[...], preferred_element_type=jnp.float32) + b2[...], 0.0)
    x = jnp.maximum(jnp.dot(x, w3[...], preferred_element_type=jnp.float32) + b3[...], 0.0)
    out_ref[...] = jnp.dot(x, wo[...], preferred_element_type=jnp.float32) + bo[...]


def _row_spec(w):
    return pl.BlockSpec((_BM, w), lambda i: (i, 0))


def _rep_spec(shape):
    return pl.BlockSpec(shape, lambda i: (0, 0))


def kernel(tree_x, edge_index, batch, W_g1, b_g1, W_g2, b_g2, W_g3, b_g3,
           W_f1, b_f1, W_f2, b_f2, W_f3, b_f3, W_out, b_out):
    src = edge_index[0]
    dst = edge_index[1]
    pad = E_PAD - N_EDGES
    src_p = jnp.concatenate([src, jnp.zeros((pad,), jnp.int32)]).reshape(NS * NB, EB)
    dst_p = jnp.concatenate([dst, jnp.full((pad,), N_NODES, jnp.int32)]).reshape(NS * NB, EB)
    ones_seed = jnp.ones((ACC_ROWS, EB), jnp.float32)
    d0, d1 = _deg_call(dst_p, ones_seed)
    x_pad = jnp.pad(tree_x, ((0, N_PAD - N_NODES), (0, 0)))

    dinv, z1 = pl.pallas_call(
        _prescale_body,
        grid=(N_PAD // _BM,),
        in_specs=[_row_spec(1), _row_spec(1), _row_spec(D_IN)],
        out_specs=[_row_spec(1), _row_spec(D_IN)],
        out_shape=[
            jax.ShapeDtypeStruct((N_PAD, 1), jnp.float32),
            jax.ShapeDtypeStruct((N_PAD, D_IN), jnp.float32),
        ],
    )(d0[:, :1], d1[:, :1], x_pad)

    g1a, g1b = _agg1(src_p, dst_p, z1)

    z2 = pl.pallas_call(
        _mm1_body,
        grid=(N_PAD // _BM,),
        in_specs=[_row_spec(1), _row_spec(128), _row_spec(128), _row_spec(128),
                  _rep_spec((D_IN, D_H)), _rep_spec((1, D_H))],
        out_specs=[_row_spec(128)] * 4,
        out_shape=[jax.ShapeDtypeStruct((N_PAD, 128), jnp.float32)] * 4,
    )(dinv, g1a, g1b, z1, W_g1, b_g1.reshape(1, D_H))

    g2 = _agg128(src_p, dst_p, *z2)

    def mm_mid(chunks, W, b):
        return pl.pallas_call(
            _mm_mid_body,
            grid=(N_PAD // _BM,),
            in_specs=[_row_spec(1)] + [_row_spec(128)] * 4
                     + [_rep_spec((D_H, D_H)), _rep_spec((1, D_H))],
            out_specs=[_row_spec(128)] * 4,
            out_shape=[jax.ShapeDtypeStruct((N_PAD, 128), jnp.float32)] * 4,
        )(dinv, *chunks, W, b.reshape(1, D_H))

    z3 = mm_mid(g2, W_g2, b_g2)
    g3 = _agg128(src_p, dst_p, *z3)

    h3 = pl.pallas_call(
        _mm3_body,
        grid=(N_PAD // _BM,),
        in_specs=[_row_spec(1)] + [_row_spec(128)] * 4
                 + [_rep_spec((D_H, D_H)), _rep_spec((1, D_H))],
        out_specs=[_row_spec(128)] * 4,
        out_shape=[jax.ShapeDtypeStruct((N_PAD, 128), jnp.float32)] * 4,
    )(dinv, *g3, W_g3, b_g3.reshape(1, D_H))

    batch_pad = jnp.pad(batch, (0, N_PAD - N_NODES), constant_values=N_GRAPHS)
    p0, p1 = _pool_call(*h3, batch_pad)
    pooled = jnp.concatenate(
        [p0.reshape(N_GRAPHS, 256), p1.reshape(N_GRAPHS, 256)], axis=1)

    return pl.pallas_call(
        _head_body,
        out_shape=jax.ShapeDtypeStruct((N_GRAPHS, N_ACTIONS), jnp.float32),
    )(pooled, W_f1, b_f1.reshape(1, D_H), W_f2, b_f2.reshape(1, D_H),
      W_f3, b_f3.reshape(1, D_H), W_out, b_out.reshape(1, N_ACTIONS))


# speedup vs baseline: 6.8249x; 1.0046x over previous
import functools
import jax
import jax.numpy as jnp
from jax import lax
from jax.experimental import pallas as pl
from jax.experimental.pallas import tpu as pltpu
from jax.experimental.pallas import tpu_sc as plsc

N_NODES = 10000
N_EDGES = 320000
N_GRAPHS = 64
D_IN = 128
D_H = 512
N_ACTIONS = 32

NC = 2
NS = 16
EB = 128
NB = 160
E_PER_TILE = NB * EB
E_PAD = NS * E_PER_TILE
N_PAD = 10240
ACC_ROWS = N_PAD
ROWS_PER_TILE = N_PAD // NS

_MESH = plsc.VectorSubcoreMesh(core_axis_name="c", subcore_axis_name="s")


HB = NB // 2


def _deg_body(dst_hbm, ones_hbm, d0_hbm, d1_hbm, di, ones_v, acc):
    c = lax.axis_index("c")
    s = lax.axis_index("s")
    pltpu.sync_copy(dst_hbm.at[pl.ds((c * NS + s) * HB, HB)], di)
    pltpu.sync_copy(ones_hbm.at[pl.ds(0, EB)], ones_v)
    pltpu.sync_copy(ones_hbm.at[pl.ds(s * ROWS_PER_TILE, ROWS_PER_TILE)],
                    acc.at[pl.ds(s * ROWS_PER_TILE, ROWS_PER_TILE)])
    plsc.subcore_barrier()

    def step(b, _):
        pltpu.sync_copy(ones_v, acc.at[di.at[b]], add=True)
        return 0

    lax.fori_loop(0, HB, step, 0)
    plsc.subcore_barrier()
    for cc in range(NC):
        @pl.when(c == cc)
        def _(cc=cc):
            out = (d0_hbm, d1_hbm)[cc]
            pltpu.sync_copy(acc.at[pl.ds(s * ROWS_PER_TILE, ROWS_PER_TILE)],
                            out.at[pl.ds(s * ROWS_PER_TILE, ROWS_PER_TILE)])


_deg_call = functools.partial(
    pl.kernel,
    out_type=[jax.ShapeDtypeStruct((N_PAD, EB), jnp.float32)] * 2,
    mesh=_MESH,
    scratch_types=[
        pltpu.VMEM((HB, EB), jnp.int32),
        pltpu.VMEM((EB, EB), jnp.float32),
        pltpu.VMEM_SHARED((ACC_ROWS, EB), jnp.float32),
    ],
)(_deg_body)


IB = 16


def _edge_scan(feat_hbm, src_hbm, dst_hbm, base, nb, si, di,
               bufA, bufB, acc, semA, semB, semSA, semSB):
    def blk(q, _):
        row0 = base + q * IB
        pltpu.sync_copy(src_hbm.at[pl.ds(row0, IB)], si)
        pltpu.sync_copy(dst_hbm.at[pl.ds(row0, IB)], di)
        pltpu.async_copy(feat_hbm.at[si.at[0]], bufA, semA)
        pltpu.async_copy(feat_hbm.at[si.at[1]], bufB, semB)
        for j in range(IB // 2):
            pltpu.make_async_copy(feat_hbm.at[si.at[2 * j]], bufA, semA).wait()
            pltpu.async_copy(bufA, acc.at[di.at[2 * j]], semSA, add=True)
            pltpu.make_async_copy(feat_hbm.at[si.at[2 * j + 1]], bufB, semB).wait()
            pltpu.async_copy(bufB, acc.at[di.at[2 * j + 1]], semSB, add=True)
            if j < IB // 2 - 1:
                pltpu.make_async_copy(bufA, acc.at[di.at[2 * j]], semSA).wait()
                pltpu.async_copy(feat_hbm.at[si.at[2 * j + 2]], bufA, semA)
                pltpu.make_async_copy(bufB, acc.at[di.at[2 * j + 1]], semSB).wait()
                pltpu.async_copy(feat_hbm.at[si.at[2 * j + 3]], bufB, semB)
            else:
                pltpu.make_async_copy(bufA, acc.at[di.at[2 * j]], semSA).wait()
                pltpu.make_async_copy(bufB, acc.at[di.at[2 * j + 1]], semSB).wait()
        return 0

    lax.fori_loop(0, nb // IB, blk, 0)


def _seed(src_hbm, dst_ref, s):
    pltpu.sync_copy(src_hbm.at[pl.ds(s * ROWS_PER_TILE, ROWS_PER_TILE)],
                    dst_ref.at[pl.ds(s * ROWS_PER_TILE, ROWS_PER_TILE)])


def _writeout(acc, out_hbm, s):
    pltpu.sync_copy(acc.at[pl.ds(s * ROWS_PER_TILE, ROWS_PER_TILE)],
                    out_hbm.at[pl.ds(s * ROWS_PER_TILE, ROWS_PER_TILE)])


_AGG_SCRATCH = [
    pltpu.VMEM((IB, EB), jnp.int32),
    pltpu.VMEM((IB, EB), jnp.int32),
    pltpu.VMEM((EB, EB), jnp.float32),
    pltpu.VMEM((EB, EB), jnp.float32),
    pltpu.VMEM_SHARED((ACC_ROWS, EB), jnp.float32),
    pltpu.SemaphoreType.DMA,
    pltpu.SemaphoreType.DMA,
    pltpu.SemaphoreType.DMA,
    pltpu.SemaphoreType.DMA,
]


def _agg128_body(src_hbm, dst_hbm, z0, z1, z2, z3, o0, o1, o2, o3,
                 si, di, bufA, bufB, acc, semA, semB, semSA, semSB):
    c = lax.axis_index("c")
    s = lax.axis_index("s")
    ins = (z0, z1, z2, z3)
    outs = (o0, o1, o2, o3)
    for cc in range(NC):
        @pl.when(c == cc)
        def _(cc=cc):
            for k in range(2):
                a = cc * 2 + k
                _seed(ins[a], acc, s)
                plsc.subcore_barrier()
                _edge_scan(ins[a], src_hbm, dst_hbm, s * NB, NB,
                           si, di, bufA, bufB, acc, semA, semB, semSA, semSB)
                plsc.subcore_barrier()
                _writeout(acc, outs[a], s)
                plsc.subcore_barrier()


_agg128 = functools.partial(
    pl.kernel,
    out_type=[jax.ShapeDtypeStruct((N_PAD, EB), jnp.float32)] * 4,
    mesh=_MESH,
    scratch_types=_AGG_SCRATCH,
)(_agg128_body)


def _agg1_body(src_hbm, dst_hbm, z_hbm, o0_hbm, o1_hbm,
               si, di, bufA, bufB, acc, semA, semB, semSA, semSB):
    c = lax.axis_index("c")
    s = lax.axis_index("s")
    _seed(z_hbm, acc, s)
    plsc.subcore_barrier()
    _edge_scan(z_hbm, src_hbm, dst_hbm, (c * NS + s) * HB, HB,
               si, di, bufA, bufB, acc, semA, semB, semSA, semSB)
    plsc.subcore_barrier()
    for cc in range(NC):
        @pl.when(c == cc)
        def _(cc=cc):
            _writeout(acc, (o0_hbm, o1_hbm)[cc], s)


_agg1 = functools.partial(
    pl.kernel,
    out_type=[jax.ShapeDtypeStruct((N_PAD, EB), jnp.float32)] * 2,
    mesh=_MESH,
    scratch_types=_AGG_SCRATCH,
)(_agg1_body)


G_ACC = 72
SB = 64
NSB = ROWS_PER_TILE // SB
_I32MIN = -2147483648


def _pool_body(c00, c01, c10, c11, batch_hbm, out0, out1,
               b_v, r0, r1, acc, tmp, res, spm):
    c = lax.axis_index("c")
    s = lax.axis_index("s")
    pltpu.sync_copy(batch_hbm.at[pl.ds(s * ROWS_PER_TILE, ROWS_PER_TILE)], b_v)
    neg = jnp.full((16,), -jnp.inf, jnp.float32)

    def ini(i, _):
        acc[pl.ds(i * 16, 16)] = neg
        return 0

    lax.fori_loop(0, G_ACC * 256 // 16, ini, 0)

    for cc in range(NC):
        @pl.when(c == cc)
        def _(cc=cc):
            h0, h1 = ((c00, c01), (c10, c11))[cc]

            def sbloop(sb, _):
                row0 = s * ROWS_PER_TILE + sb * SB
                pltpu.sync_copy(h0.at[pl.ds(row0, SB)], r0)
                pltpu.sync_copy(h1.at[pl.ds(row0, SB)], r1)
                for grp in range(SB // 16):
                    bv = b_v[pl.ds(sb * SB + grp * 16, 16)]
                    for lane in range(16):
                        r = grp * 16 + lane
                        g = bv[lane]
                        base = g * 256
                        for k in range(8):
                            o = base + k * 16
                            acc[pl.ds(o, 16)] = jnp.maximum(
                                acc[pl.ds(o, 16)], r0[r, pl.ds(k * 16, 16)])
                        for k in range(8):
                            o = base + 128 + k * 16
                            acc[pl.ds(o, 16)] = jnp.maximum(
                                acc[pl.ds(o, 16)], r1[r, pl.ds(k * 16, 16)])
                return 0

            lax.fori_loop(0, NSB, sbloop, 0)
    pltpu.sync_copy(acc, spm.at[s])
    plsc.subcore_barrier()

    @pl.when(s < 8)
    def _():
        def rini(i, _):
            res[pl.ds(i * 16, 16)] = neg
            return 0

        lax.fori_loop(0, 2048 // 16, rini, 0)
        for s2 in range(NS):
            pltpu.sync_copy(spm.at[s2, pl.ds(s * 2048, 2048)], tmp)

            def mx(i, _):
                res[pl.ds(i * 16, 16)] = jnp.maximum(
                    res[pl.ds(i * 16, 16)], tmp[pl.ds(i * 16, 16)])
                return 0

            lax.fori_loop(0, 2048 // 16, mx, 0)
        for cc in range(NC):
            @pl.when(c == cc)
            def _(cc=cc):
                out = (out0, out1)[cc]
                pltpu.sync_copy(res, out.at[pl.ds(s * 2048, 2048)])


_pool_call = functools.partial(
    pl.kernel,
    out_type=[jax.ShapeDtypeStruct((N_GRAPHS * 256,), jnp.float32)] * 2,
    mesh=_MESH,
    scratch_types=[
        pltpu.VMEM((ROWS_PER_TILE,), jnp.int32),
        pltpu.VMEM((SB, EB), jnp.float32),
        pltpu.VMEM((SB, EB), jnp.float32),
        pltpu.VMEM((G_ACC * 256,), jnp.float32),
        pltpu.VMEM((2048,), jnp.float32),
        pltpu.VMEM((2048,), jnp.float32),
        pltpu.VMEM_SHARED((NS, G_ACC * 256), jnp.float32),
    ],
)(_pool_body)


_BM = 1024


def _prescale_body(d0_ref, d1_ref, x_ref, dinv_ref, z_ref):
    deg = d0_ref[...] + d1_ref[...] - 1.0
    dinv = lax.rsqrt(deg)
    dinv_ref[...] = dinv
    z_ref[...] = x_ref[...] * dinv


def _mm_mid_body(dinv_ref, a0, a1, a2, a3, w_ref, b_ref, o0, o1, o2, o3):
    dinv = dinv_ref[...]
    agg = jnp.concatenate([a0[...], a1[...], a2[...], a3[...]], axis=1)
    h = jnp.maximum(
        jnp.dot(agg * dinv, w_ref[...], preferred_element_type=jnp.float32)
        + b_ref[...], 0.0) * dinv
    o0[...] = h[:, 0:128]
    o1[...] = h[:, 128:256]
    o2[...] = h[:, 256:384]
    o3[...] = h[:, 384:512]


def _mm1_body(dinv_ref, a0_ref, a1_ref, z_ref, w_ref, b_ref, o0, o1, o2, o3):
    dinv = dinv_ref[...]
    agg = a0_ref[...] + a1_ref[...] - z_ref[...]
    h = jnp.maximum(
        jnp.dot(agg * dinv, w_ref[...], preferred_element_type=jnp.float32)
        + b_ref[...], 0.0) * dinv
    o0[...] = h[:, 0:128]
    o1[...] = h[:, 128:256]
    o2[...] = h[:, 256:384]
    o3[...] = h[:, 384:512]


def _mm3_body(dinv_ref, a0, a1, a2, a3, w_ref, b_ref, o0, o1, o2, o3):
    dinv = dinv_ref[...]
    agg = jnp.concatenate([a0[...], a1[...], a2[...], a3[...]], axis=1)
    h = jnp.maximum(
        jnp.dot(agg * dinv, w_ref[...], preferred_element_type=jnp.float32)
        + b_ref[...], 0.0)
    o0[...] = h[:, 0:128]
    o1[...] = h[:, 128:256]
    o2[...] = h[:, 256:384]
    o3[...] = h[:, 384:512]


def _head_body(x_ref, w1, b1, w2, b2, w3, b3, wo, bo, out_ref):
    x = x_ref[...]
    x = jnp.maximum(jnp.dot(x, w1[...], preferred_element_type=jnp.float32) + b1[...], 0.0)
    x = jnp.maximum(jnp.dot(x, w2[...], preferred_element_type=jnp.float32) + b2[...], 0.0)
    x = jnp.maximum(jnp.dot(x, w3[...], preferred_element_type=jnp.float32) + b3[...], 0.0)
    out_ref[...] = jnp.dot(x, wo[...], preferred_element_type=jnp.float32) + bo[...]


def _row_spec(w):
    return pl.BlockSpec((_BM, w), lambda i: (i, 0))


def _rep_spec(shape):
    return pl.BlockSpec(shape, lambda i: (0, 0))


def kernel(tree_x, edge_index, batch, W_g1, b_g1, W_g2, b_g2, W_g3, b_g3,
           W_f1, b_f1, W_f2, b_f2, W_f3, b_f3, W_out, b_out):
    src = edge_index[0]
    dst = edge_index[1]
    pad = E_PAD - N_EDGES
    src_p = jnp.concatenate([src, jnp.zeros((pad,), jnp.int32)]).reshape(NS * NB, EB)
    trash = N_NODES + jnp.arange(pad, dtype=jnp.int32) % (N_PAD - N_NODES)
    dst_p = jnp.concatenate([dst, trash]).reshape(NS * NB, EB)
    ones_seed = jnp.ones((ACC_ROWS, EB), jnp.float32)
    d0, d1 = _deg_call(dst_p, ones_seed)
    x_pad = jnp.pad(tree_x, ((0, N_PAD - N_NODES), (0, 0)))

    dinv, z1 = pl.pallas_call(
        _prescale_body,
        grid=(N_PAD // _BM,),
        in_specs=[_row_spec(1), _row_spec(1), _row_spec(D_IN)],
        out_specs=[_row_spec(1), _row_spec(D_IN)],
        out_shape=[
            jax.ShapeDtypeStruct((N_PAD, 1), jnp.float32),
            jax.ShapeDtypeStruct((N_PAD, D_IN), jnp.float32),
        ],
    )(d0[:, :1], d1[:, :1], x_pad)

    g1a, g1b = _agg1(src_p, dst_p, z1)

    z2 = pl.pallas_call(
        _mm1_body,
        grid=(N_PAD // _BM,),
        in_specs=[_row_spec(1), _row_spec(128), _row_spec(128), _row_spec(128),
                  _rep_spec((D_IN, D_H)), _rep_spec((1, D_H))],
        out_specs=[_row_spec(128)] * 4,
        out_shape=[jax.ShapeDtypeStruct((N_PAD, 128), jnp.float32)] * 4,
    )(dinv, g1a, g1b, z1, W_g1, b_g1.reshape(1, D_H))

    g2 = _agg128(src_p, dst_p, *z2)

    def mm_mid(chunks, W, b):
        return pl.pallas_call(
            _mm_mid_body,
            grid=(N_PAD // _BM,),
            in_specs=[_row_spec(1)] + [_row_spec(128)] * 4
                     + [_rep_spec((D_H, D_H)), _rep_spec((1, D_H))],
            out_specs=[_row_spec(128)] * 4,
            out_shape=[jax.ShapeDtypeStruct((N_PAD, 128), jnp.float32)] * 4,
        )(dinv, *chunks, W, b.reshape(1, D_H))

    z3 = mm_mid(g2, W_g2, b_g2)
    g3 = _agg128(src_p, dst_p, *z3)

    h3 = pl.pallas_call(
        _mm3_body,
        grid=(N_PAD // _BM,),
        in_specs=[_row_spec(1)] + [_row_spec(128)] * 4
                 + [_rep_spec((D_H, D_H)), _rep_spec((1, D_H))],
        out_specs=[_row_spec(128)] * 4,
        out_shape=[jax.ShapeDtypeStruct((N_PAD, 128), jnp.float32)] * 4,
    )(dinv, *g3, W_g3, b_g3.reshape(1, D_H))

    batch_pad = jnp.pad(batch, (0, N_PAD - N_NODES), constant_values=N_GRAPHS)
    p0, p1 = _pool_call(*h3, batch_pad)
    pooled = jnp.concatenate(
        [p0.reshape(N_GRAPHS, 256), p1.reshape(N_GRAPHS, 256)], axis=1)

    return pl.pallas_call(
        _head_body,
        out_shape=jax.ShapeDtypeStruct((N_GRAPHS, N_ACTIONS), jnp.float32),
    )(pooled, W_f1, b_f1.reshape(1, D_H), W_f2, b_f2.reshape(1, D_H),
      W_f3, b_f3.reshape(1, D_H), W_out, b_out.reshape(1, N_ACTIONS))
